# Initial kernel scaffold; baseline (speedup 1.0000x reference)
import jax
import jax.numpy as jnp
from jax.experimental import pallas as pl


def kernel(x, adj, w, a1, a2):
    raise NotImplementedError("write your pallas kernel here")


# speedup vs baseline: 1.2183x; 1.2183x over previous
import jax
import jax.numpy as jnp
from jax.experimental import pallas as pl
from jax.experimental.pallas import tpu as pltpu

_LEAKY_ALPHA = 0.2


def _gat_fused_kernel(x_ref, adj_ref, wcat_ref, amat_ref, o_ref):
    N, D = x_ref.shape[1], x_ref.shape[2]
    H = amat_ref.shape[1] // 2

    x = x_ref[0]
    adj = adj_ref[0]
    wcat = wcat_ref[...]
    amat = amat_ref[...]

    wh_f32 = jnp.dot(x, wcat, preferred_element_type=jnp.float32)
    wh = wh_f32.astype(jnp.bfloat16)

    f_all = jnp.dot(wh, amat, preferred_element_type=jnp.float32)
    f1 = f_all[:, :H]
    f2 = f_all[:, H:]
    f2t = f2.T
    f2t_s = _LEAKY_ALPHA * f2t

    f2max = jnp.max(f2, axis=0, keepdims=True)
    m = jnp.maximum(f1 + f2max, 0.0)
    u1 = f1 - m
    v1 = _LEAKY_ALPHA * f1 - m

    adjf = adj.astype(jnp.float32)

    acc = jnp.zeros((N, D), jnp.float32)
    for h in range(H):
        u = u1[:, h:h + 1] + f2t[h:h + 1, :]
        v = v1[:, h:h + 1] + f2t_s[h:h + 1, :]
        p = jnp.exp(jnp.maximum(u, v)) * adjf
        s = jnp.sum(p, axis=-1, keepdims=True)
        r = pl.reciprocal(s, approx=True)
        y = jnp.dot(p.astype(jnp.bfloat16), wh[:, h * D:(h + 1) * D],
                    preferred_element_type=jnp.float32)
        acc = acc + y * r

    avg = acc * (1.0 / H)
    out = jnp.where(avg > 0, avg, jnp.exp(jnp.minimum(avg, 0.0)) - 1.0)
    o_ref[0] = out.astype(o_ref.dtype)


@jax.jit
def _gat_layer(x, adj, w, a1, a2):
    B, N, D = x.shape
    H = w.shape[0]

    wcat = jnp.transpose(w, (1, 0, 2)).reshape(D, H * D).astype(jnp.bfloat16)

    eye = jnp.eye(H, dtype=jnp.float32)
    a1b = (a1.reshape(H, D)[:, :, None] * eye[:, None, :]).reshape(H * D, H)
    a2b = (a2.reshape(H, D)[:, :, None] * eye[:, None, :]).reshape(H * D, H)
    amat = jnp.concatenate([a1b, a2b], axis=1).astype(jnp.bfloat16)

    xb = x.astype(jnp.bfloat16)
    adjb = adj.astype(jnp.bfloat16)

    return pl.pallas_call(
        _gat_fused_kernel,
        out_shape=jax.ShapeDtypeStruct((B, N, D), x.dtype),
        grid=(B,),
        in_specs=[
            pl.BlockSpec((1, N, D), lambda b: (b, 0, 0)),
            pl.BlockSpec((1, N, N), lambda b: (b, 0, 0)),
            pl.BlockSpec((D, H * D), lambda b: (0, 0)),
            pl.BlockSpec((H * D, 2 * H), lambda b: (0, 0)),
        ],
        out_specs=pl.BlockSpec((1, N, D), lambda b: (b, 0, 0)),
        compiler_params=pltpu.CompilerParams(
            dimension_semantics=("parallel",)),
    )(xb, adjb, wcat, amat)


def kernel(x, adj, w, a1, a2):
    return _gat_layer(x, adj, w, a1, a2)
